# baseline (speedup 1.0000x reference)
import functools

import jax
import jax.numpy as jnp
from jax import lax
from jax.experimental import pallas as pl
from jax.experimental.pallas import tpu as pltpu
from jax.experimental.pallas import tpu_sc as plsc

N = 100000
B = 128
D = 128
NC = 2
NS = 16
NW = NC * NS
CH = 160
NCHUNK = N // CH
MAXK = -(-NCHUNK // NW)

_mesh = plsc.VectorSubcoreMesh(core_axis_name="c", subcore_axis_name="s")


def _wid():
    return lax.axis_index("s") * NC + lax.axis_index("c")


def _fill(h_hbm, batch_hbm, hbuf, idxbufs, hsem, isem, slot, g):
    pltpu.make_async_copy(h_hbm.at[pl.ds(g * CH, CH)],
                          hbuf.at[slot], hsem[slot]).start()
    pltpu.make_async_copy(batch_hbm.at[pl.ds(g * CH, CH)],
                          idxbufs[slot].at[pl.ds(0, CH)], isem[slot]).start()


def _fill_wait(h_hbm, batch_hbm, hbuf, idxbufs, hsem, isem, slot, g):
    pltpu.make_async_copy(h_hbm.at[pl.ds(g * CH, CH)],
                          hbuf.at[slot], hsem[slot]).wait()
    pltpu.make_async_copy(batch_hbm.at[pl.ds(g * CH, CH)],
                          idxbufs[slot].at[pl.ds(0, CH)], isem[slot]).wait()


PCH = 128
PNCHUNK = N // PCH
PTAIL = N - PNCHUNK * PCH
PMAXK = -(-PNCHUNK // NW)


def _pfill(h_hbm, batch_hbm, hbuf, idxbufs, hsem, isem, slot, g):
    pltpu.make_async_copy(h_hbm.at[pl.ds(g * PCH, PCH)],
                          hbuf.at[slot], hsem[slot]).start()
    pltpu.make_async_copy(batch_hbm.at[pl.ds(g * PCH, PCH)],
                          idxbufs[slot], isem[slot]).start()


def _pfill_wait(h_hbm, batch_hbm, hbuf, idxbufs, hsem, isem, slot, g):
    pltpu.make_async_copy(h_hbm.at[pl.ds(g * PCH, PCH)],
                          hbuf.at[slot], hsem[slot]).wait()
    pltpu.make_async_copy(batch_hbm.at[pl.ds(g * PCH, PCH)],
                          idxbufs[slot], isem[slot]).wait()


@functools.partial(
    pl.kernel,
    mesh=_mesh,
    out_type=jax.ShapeDtypeStruct((NC, B, D), jnp.float32),
    scratch_types=[
        pltpu.VMEM((2, PCH, D), jnp.float32),
        pltpu.VMEM((PCH,), jnp.int32),
        pltpu.VMEM((PCH,), jnp.int32),
        pltpu.VMEM((PTAIL,), jnp.int32),
        pltpu.VMEM((8, D), jnp.float32),
        pltpu.VMEM((B, D), jnp.float32),
        pltpu.VMEM((B,), jnp.int32),
        pltpu.VMEM((PCH + 16,), jnp.int32),
        pltpu.VMEM_SHARED((B, D), jnp.float32),
        pltpu.SemaphoreType.DMA,
        pltpu.SemaphoreType.DMA,
        pltpu.SemaphoreType.DMA,
        pltpu.SemaphoreType.DMA,
    ],
)
def _sc_pool(h_hbm, batch_hbm, out_hbm, hbuf, idx0, idx1, idxt, zbuf,
             tpool, idxid, idxpad, pool, hs0, hs1, is0, is1):
    sid = lax.axis_index("s")
    core = lax.axis_index("c")
    w = sid * NC + core
    hsem = [hs0, hs1]
    isem = [is0, is1]
    idxbuf = [idx0, idx1]

    _pfill(h_hbm, batch_hbm, hbuf, idxbuf, hsem, isem, 0, w)

    @pl.when(w + NW < PNCHUNK)
    def _():
        _pfill(h_hbm, batch_hbm, hbuf, idxbuf, hsem, isem, 1, w + NW)

    zero = jnp.zeros((16,), jnp.float32)
    for r in range(8):
        for j in range(D // 16):
            zbuf[r, pl.ds(j * 16, 16)] = zero
    pltpu.sync_copy(zbuf, pool.at[pl.ds(sid * 8, 8)])

    def zt(r, carry):
        for j in range(D // 16):
            tpool[r, pl.ds(j * 16, 16)] = zero
        return carry

    lax.fori_loop(0, B, zt, 0)
    lanes = lax.iota(jnp.int32, 16)
    for j in range(B // 16):
        idxid[pl.ds(j * 16, 16)] = lanes + 16 * j

    plsc.subcore_barrier()

    def chunk2(k2, carry):
        for sub in range(2):
            k = k2 * 2 + sub
            g = w + k * NW

            @pl.when(g < PNCHUNK)
            def _():
                _pfill_wait(h_hbm, batch_hbm, hbuf, idxbuf, hsem, isem,
                            sub, g)

                @pl.when(lax.rem(k, 3) != 0)
                def _():
                    pltpu.sync_copy(hbuf.at[sub], pool.at[idxbuf[sub]],
                                    add=True)

                @pl.when(lax.rem(k, 3) == 0)
                def _():
                    for j in range(PCH // 16):
                        idxpad[pl.ds(j * 16, 16)] = (
                            idxbuf[sub][pl.ds(j * 16, 16)])

                    @plsc.parallel_loop(0, PCH, step=1, unroll=4)
                    def row(i):
                        seg = idxpad[pl.ds(i, 16)][0]
                        for j in range(D // 16):
                            plsc.addupdate(
                                tpool.at[seg, pl.ds(j * 16, 16)],
                                hbuf[sub, i, pl.ds(j * 16, 16)])

                @pl.when(g + 2 * NW < PNCHUNK)
                def _():
                    _pfill(h_hbm, batch_hbm, hbuf, idxbuf, hsem, isem,
                           sub, g + 2 * NW)

        return carry

    lax.fori_loop(0, (PMAXK + 1) // 2, chunk2, 0)

    pltpu.sync_copy(tpool, pool.at[idxid], add=True)

    @pl.when(jnp.logical_and(sid == 0, core == 0))
    def _():
        pltpu.sync_copy(h_hbm.at[pl.ds(PNCHUNK * PCH, PTAIL)],
                        hbuf.at[0, pl.ds(0, PTAIL)])
        pltpu.sync_copy(batch_hbm.at[pl.ds(PNCHUNK * PCH, PTAIL)], idxt)
        pltpu.sync_copy(hbuf.at[0, pl.ds(0, PTAIL)], pool.at[idxt],
                        add=True)

    plsc.subcore_barrier()

    @pl.when(sid == 0)
    def _():
        pltpu.sync_copy(pool, out_hbm.at[core])


@functools.partial(
    pl.kernel,
    mesh=_mesh,
    out_type=jax.ShapeDtypeStruct((N, D), jnp.float32),
    scratch_types=[
        pltpu.VMEM((4, CH, D), jnp.float32),
        pltpu.VMEM((CH + 16,), jnp.int32),
        pltpu.VMEM((CH + 16,), jnp.int32),
        pltpu.VMEM((CH + 16,), jnp.int32),
        pltpu.VMEM((CH + 16,), jnp.int32),
        pltpu.VMEM((B, D), jnp.float32),
        pltpu.SemaphoreType.DMA,
        pltpu.SemaphoreType.DMA,
        pltpu.SemaphoreType.DMA,
        pltpu.SemaphoreType.DMA,
        pltpu.SemaphoreType.DMA,
        pltpu.SemaphoreType.DMA,
        pltpu.SemaphoreType.DMA,
        pltpu.SemaphoreType.DMA,
        pltpu.SemaphoreType.DMA,
        pltpu.SemaphoreType.DMA,
        pltpu.SemaphoreType.DMA,
        pltpu.SemaphoreType.DMA,
    ],
)
def _sc_bcast(h_hbm, batch_hbm, vn_hbm, out_hbm, hbuf,
              idx0, idx1, idx2, idx3, vnbuf,
              hs0, hs1, hs2, hs3, is0, is1, is2, is3, os0, os1, os2, os3):
    w = _wid()
    hsem = [hs0, hs1, hs2, hs3]
    isem = [is0, is1, is2, is3]
    osem = [os0, os1, os2, os3]
    idxbuf = [idx0, idx1, idx2, idx3]

    def store(slot, g):
        pltpu.make_async_copy(hbuf.at[slot],
                              out_hbm.at[pl.ds(g * CH, CH)],
                              osem[slot]).start()

    def store_wait(slot, g):
        pltpu.make_async_copy(hbuf.at[slot],
                              out_hbm.at[pl.ds(g * CH, CH)],
                              osem[slot]).wait()

    _fill(h_hbm, batch_hbm, hbuf, idxbuf, hsem, isem, 0, w)
    _fill(h_hbm, batch_hbm, hbuf, idxbuf, hsem, isem, 1, w + NW)
    pltpu.sync_copy(vn_hbm, vnbuf)

    def chunk4(k4, carry):
        for sub in range(4):
            k = k4 * 4 + sub
            g = w + k * NW

            @pl.when(g < NCHUNK)
            def _():
                @pl.when(k >= 2)
                def _():
                    store_wait((sub + 2) % 4, g - 2 * NW)

                @pl.when(g + 2 * NW < NCHUNK)
                def _():
                    _fill(h_hbm, batch_hbm, hbuf, idxbuf, hsem, isem,
                          (sub + 2) % 4, g + 2 * NW)

                _fill_wait(h_hbm, batch_hbm, hbuf, idxbuf, hsem, isem,
                           sub, g)

                @plsc.parallel_loop(0, CH, step=1, unroll=4)
                def row(i):
                    seg = idxbuf[sub][pl.ds(i, 16)][0]
                    for j in range(D // 16):
                        plsc.addupdate(hbuf.at[sub, i, pl.ds(j * 16, 16)],
                                       vnbuf[seg, pl.ds(j * 16, 16)])

                store(sub, g)

        return carry

    lax.fori_loop(0, MAXK // 4, chunk4, 0)

    nk = (NCHUNK - 1 - w) // NW + 1
    for back in (2, 1):
        kk = nk - back
        for sub in range(4):
            @pl.when(jnp.logical_and(kk >= 0, lax.rem(kk, 4) == sub))
            def _():
                store_wait(sub, w + kk * NW)


def _mlp_kernel(part_ref, vnh_ref, w_ref, bias_ref, vnout_ref):
    pool = jnp.sum(part_ref[...], axis=0)
    x = vnh_ref[...] + pool
    t = lax.dot_general(x, w_ref[...], (((1,), (0,)), ((), ())),
                        preferred_element_type=jnp.float32)
    vnout_ref[...] = vnh_ref[...] + jnp.maximum(t + bias_ref[...], 0.0)


@jax.jit
def kernel(h, batch, vn_h, W, b):
    batch32 = batch.astype(jnp.int32)
    bias2 = b.reshape(1, D)

    partials = _sc_pool(h, batch32)

    vn_new = pl.pallas_call(
        _mlp_kernel,
        out_shape=jax.ShapeDtypeStruct((B, D), jnp.float32),
    )(partials, vn_h, W, bias2)

    h_new = _sc_bcast(h, batch32, vn_new)

    return h_new, vn_new


# speedup vs baseline: 1.0524x; 1.0524x over previous
import functools

import jax
import jax.numpy as jnp
from jax import lax
from jax.experimental import pallas as pl
from jax.experimental.pallas import tpu as pltpu
from jax.experimental.pallas import tpu_sc as plsc

N = 100000
B = 128
D = 128
NC = 2
NS = 16
NW = NC * NS
CH = 160
NCHUNK = N // CH
MAXK = -(-NCHUNK // NW)

_mesh = plsc.VectorSubcoreMesh(core_axis_name="c", subcore_axis_name="s")


def _wid():
    return lax.axis_index("s") * NC + lax.axis_index("c")


def _fill(h_hbm, batch_hbm, hbuf, idxbufs, hsem, isem, slot, g):
    pltpu.make_async_copy(h_hbm.at[pl.ds(g * CH, CH)],
                          hbuf.at[slot], hsem[slot]).start()
    pltpu.make_async_copy(batch_hbm.at[pl.ds(g * CH, CH)],
                          idxbufs[slot].at[pl.ds(0, CH)], isem[slot]).start()


def _fill_wait(h_hbm, batch_hbm, hbuf, idxbufs, hsem, isem, slot, g):
    pltpu.make_async_copy(h_hbm.at[pl.ds(g * CH, CH)],
                          hbuf.at[slot], hsem[slot]).wait()
    pltpu.make_async_copy(batch_hbm.at[pl.ds(g * CH, CH)],
                          idxbufs[slot].at[pl.ds(0, CH)], isem[slot]).wait()


PCH = 128
PNCHUNK = N // PCH
PTAIL = N - PNCHUNK * PCH
PMAXK = -(-PNCHUNK // NW)


def _pfill(h_hbm, batch_hbm, hbuf, idxbufs, hsem, isem, slot, g):
    pltpu.make_async_copy(h_hbm.at[pl.ds(g * PCH, PCH)],
                          hbuf.at[slot], hsem[slot]).start()
    pltpu.make_async_copy(batch_hbm.at[pl.ds(g * PCH, PCH)],
                          idxbufs[slot], isem[slot]).start()


def _pfill_wait(h_hbm, batch_hbm, hbuf, idxbufs, hsem, isem, slot, g):
    pltpu.make_async_copy(h_hbm.at[pl.ds(g * PCH, PCH)],
                          hbuf.at[slot], hsem[slot]).wait()
    pltpu.make_async_copy(batch_hbm.at[pl.ds(g * PCH, PCH)],
                          idxbufs[slot], isem[slot]).wait()


@functools.partial(
    pl.kernel,
    mesh=_mesh,
    out_type=jax.ShapeDtypeStruct((NC, B, D), jnp.float32),
    scratch_types=[
        pltpu.VMEM((2, PCH, D), jnp.float32),
        pltpu.VMEM((PCH,), jnp.int32),
        pltpu.VMEM((PCH,), jnp.int32),
        pltpu.VMEM((PTAIL,), jnp.int32),
        pltpu.VMEM((8, D), jnp.float32),
        pltpu.VMEM_SHARED((B, D), jnp.float32),
        pltpu.SemaphoreType.DMA,
        pltpu.SemaphoreType.DMA,
        pltpu.SemaphoreType.DMA,
        pltpu.SemaphoreType.DMA,
    ],
)
def _sc_pool(h_hbm, batch_hbm, out_hbm, hbuf, idx0, idx1, idxt, zbuf, pool,
             hs0, hs1, is0, is1):
    sid = lax.axis_index("s")
    core = lax.axis_index("c")
    w = sid * NC + core
    hsem = [hs0, hs1]
    isem = [is0, is1]
    idxbuf = [idx0, idx1]

    _pfill(h_hbm, batch_hbm, hbuf, idxbuf, hsem, isem, 0, w)

    @pl.when(w + NW < PNCHUNK)
    def _():
        _pfill(h_hbm, batch_hbm, hbuf, idxbuf, hsem, isem, 1, w + NW)

    zero = jnp.zeros((16,), jnp.float32)
    for r in range(8):
        for j in range(D // 16):
            zbuf[r, pl.ds(j * 16, 16)] = zero
    pltpu.sync_copy(zbuf, pool.at[pl.ds(sid * 8, 8)])
    plsc.subcore_barrier()

    def chunk2(k2, carry):
        for sub in range(2):
            k = k2 * 2 + sub
            g = w + k * NW

            @pl.when(g < PNCHUNK)
            def _():
                _pfill_wait(h_hbm, batch_hbm, hbuf, idxbuf, hsem, isem,
                            sub, g)
                pltpu.sync_copy(hbuf.at[sub], pool.at[idxbuf[sub]],
                                add=True)

                @pl.when(g + 2 * NW < PNCHUNK)
                def _():
                    _pfill(h_hbm, batch_hbm, hbuf, idxbuf, hsem, isem,
                           sub, g + 2 * NW)

        return carry

    lax.fori_loop(0, (PMAXK + 1) // 2, chunk2, 0)

    @pl.when(jnp.logical_and(sid == 0, core == 0))
    def _():
        pltpu.sync_copy(h_hbm.at[pl.ds(PNCHUNK * PCH, PTAIL)],
                        hbuf.at[0, pl.ds(0, PTAIL)])
        pltpu.sync_copy(batch_hbm.at[pl.ds(PNCHUNK * PCH, PTAIL)], idxt)
        pltpu.sync_copy(hbuf.at[0, pl.ds(0, PTAIL)], pool.at[idxt],
                        add=True)

    plsc.subcore_barrier()

    @pl.when(sid == 0)
    def _():
        pltpu.sync_copy(pool, out_hbm.at[core])


@functools.partial(
    pl.kernel,
    mesh=_mesh,
    out_type=jax.ShapeDtypeStruct((N, D), jnp.float32),
    scratch_types=[
        pltpu.VMEM((4, CH, D), jnp.float32),
        pltpu.VMEM((CH + 16,), jnp.int32),
        pltpu.VMEM((CH + 16,), jnp.int32),
        pltpu.VMEM((CH + 16,), jnp.int32),
        pltpu.VMEM((CH + 16,), jnp.int32),
        pltpu.VMEM((B, D), jnp.float32),
        pltpu.SemaphoreType.DMA,
        pltpu.SemaphoreType.DMA,
        pltpu.SemaphoreType.DMA,
        pltpu.SemaphoreType.DMA,
        pltpu.SemaphoreType.DMA,
        pltpu.SemaphoreType.DMA,
        pltpu.SemaphoreType.DMA,
        pltpu.SemaphoreType.DMA,
        pltpu.SemaphoreType.DMA,
        pltpu.SemaphoreType.DMA,
        pltpu.SemaphoreType.DMA,
        pltpu.SemaphoreType.DMA,
    ],
)
def _sc_bcast(h_hbm, batch_hbm, vn_hbm, out_hbm, hbuf,
              idx0, idx1, idx2, idx3, vnbuf,
              hs0, hs1, hs2, hs3, is0, is1, is2, is3, os0, os1, os2, os3):
    w = _wid()
    hsem = [hs0, hs1, hs2, hs3]
    isem = [is0, is1, is2, is3]
    osem = [os0, os1, os2, os3]
    idxbuf = [idx0, idx1, idx2, idx3]

    def store(slot, g):
        pltpu.make_async_copy(hbuf.at[slot],
                              out_hbm.at[pl.ds(g * CH, CH)],
                              osem[slot]).start()

    def store_wait(slot, g):
        pltpu.make_async_copy(hbuf.at[slot],
                              out_hbm.at[pl.ds(g * CH, CH)],
                              osem[slot]).wait()

    _fill(h_hbm, batch_hbm, hbuf, idxbuf, hsem, isem, 0, w)
    _fill(h_hbm, batch_hbm, hbuf, idxbuf, hsem, isem, 1, w + NW)
    pltpu.sync_copy(vn_hbm, vnbuf)

    def chunk4(k4, carry):
        for sub in range(4):
            k = k4 * 4 + sub
            g = w + k * NW

            @pl.when(g < NCHUNK)
            def _():
                @pl.when(k >= 2)
                def _():
                    store_wait((sub + 2) % 4, g - 2 * NW)

                @pl.when(g + 2 * NW < NCHUNK)
                def _():
                    _fill(h_hbm, batch_hbm, hbuf, idxbuf, hsem, isem,
                          (sub + 2) % 4, g + 2 * NW)

                _fill_wait(h_hbm, batch_hbm, hbuf, idxbuf, hsem, isem,
                           sub, g)

                @plsc.parallel_loop(0, CH, step=1, unroll=4)
                def row(i):
                    seg = idxbuf[sub][pl.ds(i, 16)][0]
                    for j in range(D // 16):
                        plsc.addupdate(hbuf.at[sub, i, pl.ds(j * 16, 16)],
                                       vnbuf[seg, pl.ds(j * 16, 16)])

                store(sub, g)

        return carry

    lax.fori_loop(0, MAXK // 4, chunk4, 0)

    nk = (NCHUNK - 1 - w) // NW + 1
    for back in (2, 1):
        kk = nk - back
        for sub in range(4):
            @pl.when(jnp.logical_and(kk >= 0, lax.rem(kk, 4) == sub))
            def _():
                store_wait(sub, w + kk * NW)


def _mlp_kernel(part_ref, vnh_ref, w_ref, bias_ref, vnout_ref):
    pool = jnp.sum(part_ref[...], axis=0)
    x = vnh_ref[...] + pool
    t = lax.dot_general(x, w_ref[...], (((1,), (0,)), ((), ())),
                        preferred_element_type=jnp.float32)
    vnout_ref[...] = vnh_ref[...] + jnp.maximum(t + bias_ref[...], 0.0)


@jax.jit
def kernel(h, batch, vn_h, W, b):
    batch32 = batch.astype(jnp.int32)
    bias2 = b.reshape(1, D)

    partials = _sc_pool(h, batch32)

    vn_new = pl.pallas_call(
        _mlp_kernel,
        out_shape=jax.ShapeDtypeStruct((B, D), jnp.float32),
    )(partials, vn_h, W, bias2)

    h_new = _sc_bcast(h, batch32, vn_new)

    return h_new, vn_new


# speedup vs baseline: 1.0759x; 1.0224x over previous
import functools

import jax
import jax.numpy as jnp
from jax import lax
from jax.experimental import pallas as pl
from jax.experimental.pallas import tpu as pltpu
from jax.experimental.pallas import tpu_sc as plsc

N = 100000
B = 128
D = 128
NC = 2
NS = 16
NW = NC * NS
CH = 160
NCHUNK = N // CH
MAXK = -(-NCHUNK // NW)

_mesh = plsc.VectorSubcoreMesh(core_axis_name="c", subcore_axis_name="s")


def _wid():
    return lax.axis_index("s") * NC + lax.axis_index("c")


def _fill(h_hbm, batch_hbm, hbuf, idxbufs, hsem, isem, slot, g):
    pltpu.make_async_copy(h_hbm.at[pl.ds(g * CH, CH)],
                          hbuf.at[slot], hsem[slot]).start()
    pltpu.make_async_copy(batch_hbm.at[pl.ds(g * CH, CH)],
                          idxbufs[slot].at[pl.ds(0, CH)], isem[slot]).start()


def _fill_wait(h_hbm, batch_hbm, hbuf, idxbufs, hsem, isem, slot, g):
    pltpu.make_async_copy(h_hbm.at[pl.ds(g * CH, CH)],
                          hbuf.at[slot], hsem[slot]).wait()
    pltpu.make_async_copy(batch_hbm.at[pl.ds(g * CH, CH)],
                          idxbufs[slot].at[pl.ds(0, CH)], isem[slot]).wait()


PCH = 128
PNCHUNK = N // PCH
PTAIL = N - PNCHUNK * PCH
PMAXK = -(-PNCHUNK // NW)


def _pfill(h_hbm, batch_hbm, hbuf, idxbufs, hsem, isem, slot, g):
    pltpu.make_async_copy(h_hbm.at[pl.ds(g * PCH, PCH)],
                          hbuf.at[slot], hsem[slot]).start()
    pltpu.make_async_copy(batch_hbm.at[pl.ds(g * PCH, PCH)],
                          idxbufs[slot], isem[slot]).start()


def _pfill_wait(h_hbm, batch_hbm, hbuf, idxbufs, hsem, isem, slot, g):
    pltpu.make_async_copy(h_hbm.at[pl.ds(g * PCH, PCH)],
                          hbuf.at[slot], hsem[slot]).wait()
    pltpu.make_async_copy(batch_hbm.at[pl.ds(g * PCH, PCH)],
                          idxbufs[slot], isem[slot]).wait()


@functools.partial(
    pl.kernel,
    mesh=_mesh,
    out_type=jax.ShapeDtypeStruct((NC, B, D), jnp.float32),
    scratch_types=[
        pltpu.VMEM((4, PCH, D), jnp.float32),
        pltpu.VMEM((PCH,), jnp.int32),
        pltpu.VMEM((PCH,), jnp.int32),
        pltpu.VMEM((PCH,), jnp.int32),
        pltpu.VMEM((PCH,), jnp.int32),
        pltpu.VMEM((PTAIL,), jnp.int32),
        pltpu.VMEM((8, D), jnp.float32),
        pltpu.VMEM_SHARED((B, D), jnp.float32),
        pltpu.SemaphoreType.DMA,
        pltpu.SemaphoreType.DMA,
        pltpu.SemaphoreType.DMA,
        pltpu.SemaphoreType.DMA,
        pltpu.SemaphoreType.DMA,
        pltpu.SemaphoreType.DMA,
        pltpu.SemaphoreType.DMA,
        pltpu.SemaphoreType.DMA,
    ],
)
def _sc_pool(h_hbm, batch_hbm, out_hbm, hbuf, idx0, idx1, idx2, idx3,
             idxt, zbuf, pool, hs0, hs1, hs2, hs3, is0, is1, is2, is3):
    sid = lax.axis_index("s")
    core = lax.axis_index("c")
    w = sid * NC + core
    hsem = [hs0, hs1, hs2, hs3]
    isem = [is0, is1, is2, is3]
    idxbuf = [idx0, idx1, idx2, idx3]

    _pfill(h_hbm, batch_hbm, hbuf, idxbuf, hsem, isem, 0, w)

    @pl.when(w + NW < PNCHUNK)
    def _():
        _pfill(h_hbm, batch_hbm, hbuf, idxbuf, hsem, isem, 1, w + NW)

    zero = jnp.zeros((16,), jnp.float32)
    for r in range(8):
        for j in range(D // 16):
            zbuf[r, pl.ds(j * 16, 16)] = zero
    pltpu.sync_copy(zbuf, pool.at[pl.ds(sid * 8, 8)])
    plsc.subcore_barrier()

    def chunk4(k4, carry):
        for sub in range(4):
            k = k4 * 4 + sub
            g = w + k * NW

            @pl.when(g < PNCHUNK)
            def _():
                _pfill_wait(h_hbm, batch_hbm, hbuf, idxbuf, hsem, isem,
                            sub, g)

                @pl.when(g + 2 * NW < PNCHUNK)
                def _():
                    _pfill(h_hbm, batch_hbm, hbuf, idxbuf, hsem, isem,
                           (sub + 2) % 4, g + 2 * NW)

                pltpu.sync_copy(hbuf.at[sub], pool.at[idxbuf[sub]],
                                add=True)

        return carry

    lax.fori_loop(0, (PMAXK + 3) // 4, chunk4, 0)

    @pl.when(jnp.logical_and(sid == 0, core == 0))
    def _():
        pltpu.sync_copy(h_hbm.at[pl.ds(PNCHUNK * PCH, PTAIL)],
                        hbuf.at[0, pl.ds(0, PTAIL)])
        pltpu.sync_copy(batch_hbm.at[pl.ds(PNCHUNK * PCH, PTAIL)], idxt)
        pltpu.sync_copy(hbuf.at[0, pl.ds(0, PTAIL)], pool.at[idxt],
                        add=True)

    plsc.subcore_barrier()

    @pl.when(sid == 0)
    def _():
        pltpu.sync_copy(pool, out_hbm.at[core])


@functools.partial(
    pl.kernel,
    mesh=_mesh,
    out_type=jax.ShapeDtypeStruct((N, D), jnp.float32),
    scratch_types=[
        pltpu.VMEM((4, CH, D), jnp.float32),
        pltpu.VMEM((CH + 16,), jnp.int32),
        pltpu.VMEM((CH + 16,), jnp.int32),
        pltpu.VMEM((CH + 16,), jnp.int32),
        pltpu.VMEM((CH + 16,), jnp.int32),
        pltpu.VMEM((B, D), jnp.float32),
        pltpu.SemaphoreType.DMA,
        pltpu.SemaphoreType.DMA,
        pltpu.SemaphoreType.DMA,
        pltpu.SemaphoreType.DMA,
        pltpu.SemaphoreType.DMA,
        pltpu.SemaphoreType.DMA,
        pltpu.SemaphoreType.DMA,
        pltpu.SemaphoreType.DMA,
        pltpu.SemaphoreType.DMA,
        pltpu.SemaphoreType.DMA,
        pltpu.SemaphoreType.DMA,
        pltpu.SemaphoreType.DMA,
    ],
)
def _sc_bcast(h_hbm, batch_hbm, vn_hbm, out_hbm, hbuf,
              idx0, idx1, idx2, idx3, vnbuf,
              hs0, hs1, hs2, hs3, is0, is1, is2, is3, os0, os1, os2, os3):
    w = _wid()
    hsem = [hs0, hs1, hs2, hs3]
    isem = [is0, is1, is2, is3]
    osem = [os0, os1, os2, os3]
    idxbuf = [idx0, idx1, idx2, idx3]

    def store(slot, g):
        pltpu.make_async_copy(hbuf.at[slot],
                              out_hbm.at[pl.ds(g * CH, CH)],
                              osem[slot]).start()

    def store_wait(slot, g):
        pltpu.make_async_copy(hbuf.at[slot],
                              out_hbm.at[pl.ds(g * CH, CH)],
                              osem[slot]).wait()

    _fill(h_hbm, batch_hbm, hbuf, idxbuf, hsem, isem, 0, w)
    _fill(h_hbm, batch_hbm, hbuf, idxbuf, hsem, isem, 1, w + NW)
    pltpu.sync_copy(vn_hbm, vnbuf)

    def chunk4(k4, carry):
        for sub in range(4):
            k = k4 * 4 + sub
            g = w + k * NW

            @pl.when(g < NCHUNK)
            def _():
                @pl.when(k >= 2)
                def _():
                    store_wait((sub + 2) % 4, g - 2 * NW)

                @pl.when(g + 2 * NW < NCHUNK)
                def _():
                    _fill(h_hbm, batch_hbm, hbuf, idxbuf, hsem, isem,
                          (sub + 2) % 4, g + 2 * NW)

                _fill_wait(h_hbm, batch_hbm, hbuf, idxbuf, hsem, isem,
                           sub, g)

                @plsc.parallel_loop(0, CH, step=1, unroll=4)
                def row(i):
                    seg = idxbuf[sub][pl.ds(i, 16)][0]
                    for j in range(D // 16):
                        plsc.addupdate(hbuf.at[sub, i, pl.ds(j * 16, 16)],
                                       vnbuf[seg, pl.ds(j * 16, 16)])

                store(sub, g)

        return carry

    lax.fori_loop(0, MAXK // 4, chunk4, 0)

    nk = (NCHUNK - 1 - w) // NW + 1
    for back in (2, 1):
        kk = nk - back
        for sub in range(4):
            @pl.when(jnp.logical_and(kk >= 0, lax.rem(kk, 4) == sub))
            def _():
                store_wait(sub, w + kk * NW)


def _mlp_kernel(part_ref, vnh_ref, w_ref, bias_ref, vnout_ref):
    pool = jnp.sum(part_ref[...], axis=0)
    x = vnh_ref[...] + pool
    t = lax.dot_general(x, w_ref[...], (((1,), (0,)), ((), ())),
                        preferred_element_type=jnp.float32)
    vnout_ref[...] = vnh_ref[...] + jnp.maximum(t + bias_ref[...], 0.0)


@jax.jit
def kernel(h, batch, vn_h, W, b):
    batch32 = batch.astype(jnp.int32)
    bias2 = b.reshape(1, D)

    partials = _sc_pool(h, batch32)

    vn_new = pl.pallas_call(
        _mlp_kernel,
        out_shape=jax.ShapeDtypeStruct((B, D), jnp.float32),
    )(partials, vn_h, W, bias2)

    h_new = _sc_bcast(h, batch32, vn_new)

    return h_new, vn_new
